# baseline (speedup 1.0000x reference)
import functools

import jax
import jax.numpy as jnp
from jax import lax
from jax.experimental import pallas as pl
from jax.experimental.pallas import tpu as pltpu

L = 2048
DIM = 1024
DIN = 2048
DSTATE = 16
DTRANK = 64
DCONV = 4
NEXP = 8
FFI = 2048
LC = 32
RC = 256

_F32 = jnp.float32
_NT = (((1,), (1,)), ((), ()))


def _silu(v):
    return v * jax.nn.sigmoid(v)


def _softplus(v):
    return jnp.maximum(v, 0.0) + jnp.log1p(jnp.exp(-jnp.abs(v)))


def _ntdot(a, b):
    return lax.dot_general(a, b, _NT, preferred_element_type=_F32)


_BF16 = jnp.bfloat16


def _ntdot16(a, b16):
    return lax.dot_general(a.astype(_BF16), b16, _NT,
                           preferred_element_type=_F32)


def _mamba_body(x_ref, w_in_ref, conv_w_ref, conv_b_ref, wx_ref,
                w_dt_ref, b_dt_ref, alog_ref, dd_ref, w_out_ref,
                out_ref, tail_ref, state_ref, da_s, dbu_s, st_s,
                w_in16_s, w_out16_s):
    c = pl.program_id(0)

    @pl.when(c == 0)
    def _():
        tail_ref[...] = jnp.zeros_like(tail_ref)
        state_ref[...] = jnp.zeros_like(state_ref)
        w_in16_s[...] = w_in_ref[...].astype(_BF16)
        w_out16_s[...] = w_out_ref[...].astype(_BF16)

    xch = x_ref[...]
    xz = _ntdot16(xch, w_in16_s[...])
    xc = xz[:, :DIN]
    res = xz[:, DIN:]

    ext = jnp.concatenate([tail_ref[...], xc], axis=0)
    tail_ref[...] = xc[LC - (DCONV - 1):, :]
    conv = conv_b_ref[...]
    for k in range(DCONV):
        conv = conv + ext[k:k + LC, :] * conv_w_ref[k:k + 1, :]
    xcs = _silu(conv)

    x_dbl = _ntdot(xcs, wx_ref[...])
    delta_r = x_dbl[:, :DTRANK]
    bm = x_dbl[:, DTRANK:DTRANK + DSTATE]
    cm = x_dbl[:, DTRANK + DSTATE:]
    delta = _softplus(_ntdot(delta_r, w_dt_ref[...]) + b_dt_ref[...])
    u = delta * xcs

    a2 = -jnp.exp(alog_ref[...])
    da_s[...] = jnp.exp(delta[:, None, :] * a2[None, :, :])
    dbu_s[...] = bm[:, :, None] * u[:, None, :]

    def step(l, _):
        st = (da_s[pl.ds(l, 1)][0] * state_ref[...]
              + dbu_s[pl.ds(l, 1)][0])
        state_ref[...] = st
        st_s[pl.ds(l, 1)] = st[None]
        return 0

    lax.fori_loop(0, LC, step, 0, unroll=False)

    y = jnp.sum(st_s[...] * cm[:, :, None], axis=1)
    y = y + xcs * dd_ref[...]
    y = y * _silu(res)
    out_ref[...] = _ntdot16(y, w_out16_s[...]) + xch


def _mamba_block(h, bp):
    conv_w_t = bp['conv_w'].T
    conv_b = bp['conv_b'].reshape(1, DIN)
    b_dt = bp['b_dt'].reshape(1, DIN)
    alog_t = bp['A_log'].T
    dd = bp['D'].reshape(1, DIN)

    grid = L // LC
    full = lambda shape: pl.BlockSpec(shape, lambda c: (0,) * len(shape))
    return pl.pallas_call(
        _mamba_body,
        grid=(grid,),
        in_specs=[
            pl.BlockSpec((LC, DIM), lambda c: (c, 0)),
            full((2 * DIN, DIM)),
            full((DCONV, DIN)),
            full((1, DIN)),
            full((DTRANK + 2 * DSTATE, DIN)),
            full((DIN, DTRANK)),
            full((1, DIN)),
            full((DSTATE, DIN)),
            full((1, DIN)),
            full((DIM, DIN)),
        ],
        out_specs=pl.BlockSpec((LC, DIM), lambda c: (c, 0)),
        out_shape=jax.ShapeDtypeStruct((L, DIM), _F32),
        scratch_shapes=[
            pltpu.VMEM((DCONV - 1, DIN), _F32),
            pltpu.VMEM((DSTATE, DIN), _F32),
            pltpu.VMEM((LC, DSTATE, DIN), _F32),
            pltpu.VMEM((LC, DSTATE, DIN), _F32),
            pltpu.VMEM((LC, DSTATE, DIN), _F32),
            pltpu.VMEM((2 * DIN, DIM), _BF16),
            pltpu.VMEM((DIM, DIN), _BF16),
        ],
    )(h, bp['W_in'], conv_w_t, conv_b, bp['W_x'], bp['W_dt'], b_dt,
      alog_t, dd, bp['W_out'])


def _top2_weight(h, wgate, e):
    scores = _ntdot(h, wgate)
    ii = lax.broadcasted_iota(jnp.int32, scores.shape, 1)
    m1 = jnp.max(scores, axis=-1, keepdims=True)
    a1 = jnp.min(jnp.where(scores == m1, ii, NEXP), axis=-1, keepdims=True)
    s2 = jnp.where(ii == a1, -jnp.inf, scores)
    m2 = jnp.max(s2, axis=-1, keepdims=True)
    a2 = jnp.min(jnp.where(s2 == m2, ii, NEXP), axis=-1, keepdims=True)
    e2 = jnp.exp(m2 - m1)
    w1 = 1.0 / (1.0 + e2)
    w2 = 1.0 - w1
    return jnp.where(a1 == e, w1, 0.0) + jnp.where(a2 == e, w2, 0.0)


def _moe_exp_body(e, h_ref, acc_ref, wgate_ref, wg_ref, wu_ref, wd_ref,
                  out_ref, wg16_s, wu16_s, wd16_s):
    @pl.when(pl.program_id(0) == 0)
    def _():
        wg16_s[...] = wg_ref[...].astype(_BF16)
        wu16_s[...] = wu_ref[...].astype(_BF16)
        wd16_s[...] = wd_ref[...].astype(_BF16)

    h = h_ref[...]
    we = _top2_weight(h, wgate_ref[...], e)
    h16 = h.astype(_BF16)
    gate = _silu(lax.dot_general(h16, wg16_s[...], _NT,
                                 preferred_element_type=_F32))
    up = lax.dot_general(h16, wu16_s[...], _NT, preferred_element_type=_F32)
    ffn = _ntdot16(gate * up, wd16_s[...])
    out_ref[...] = acc_ref[...] + we * ffn


def _moe_block(h, mp):
    acc = h
    for e in range(NEXP):
        ep = mp['experts'][e]
        acc = pl.pallas_call(
            functools.partial(_moe_exp_body, e),
            grid=(L // RC,),
            in_specs=[
                pl.BlockSpec((RC, DIM), lambda r: (r, 0)),
                pl.BlockSpec((RC, DIM), lambda r: (r, 0)),
                pl.BlockSpec((NEXP, DIM), lambda r: (0, 0)),
                pl.BlockSpec((FFI, DIM), lambda r: (0, 0)),
                pl.BlockSpec((FFI, DIM), lambda r: (0, 0)),
                pl.BlockSpec((DIM, FFI), lambda r: (0, 0)),
            ],
            out_specs=pl.BlockSpec((RC, DIM), lambda r: (r, 0)),
            out_shape=jax.ShapeDtypeStruct((L, DIM), _F32),
            scratch_shapes=[
                pltpu.VMEM((FFI, DIM), _BF16),
                pltpu.VMEM((FFI, DIM), _BF16),
                pltpu.VMEM((DIM, FFI), _BF16),
            ],
        )(h, acc, mp['W_gate'], ep['Wg'], ep['Wu'], ep['Wd'])
    return acc


def _head_body(h_ref, w_ref, out_ref, w16_s):
    @pl.when(pl.program_id(0) == 0)
    def _():
        w16_s[...] = w_ref[...].astype(_BF16)
    out_ref[...] = jax.nn.sigmoid(_ntdot16(h_ref[...], w16_s[...]))


def _head(h, w_head):
    return pl.pallas_call(
        _head_body,
        grid=(L // RC,),
        in_specs=[
            pl.BlockSpec((RC, DIM), lambda r: (r, 0)),
            pl.BlockSpec((DIM, DIM), lambda r: (0, 0)),
        ],
        out_specs=pl.BlockSpec((RC, DIM), lambda r: (r, 0)),
        out_shape=jax.ShapeDtypeStruct((L, DIM), _F32),
        scratch_shapes=[pltpu.VMEM((DIM, DIM), _BF16)],
    )(h, w_head)


def kernel(x, params):
    h = x.reshape(L, DIM)
    for i in range(len(params['blocks'])):
        h = _mamba_block(h, params['blocks'][i])
        h = _moe_block(h, params['moes'][i])
    h = _head(h, params['W_head'])
    return h.reshape(x.shape)
